# baseline (speedup 1.0000x reference)
import jax
import jax.numpy as jnp
from jax.experimental import pallas as pl
from jax.experimental.pallas import tpu as pltpu


def _router_kernel(x1_ref, x2_ref, w1_ref, w2_ref, gate_ref, probs_ref, idx_ref):
    T, E = gate_ref.shape
    logits = jnp.dot(x1_ref[...], w1_ref[...], preferred_element_type=jnp.float32)
    logits = logits + jnp.dot(
        x2_ref[...], w2_ref[...], preferred_element_type=jnp.float32
    )
    gate_ref[...] = logits
    iota = jax.lax.broadcasted_iota(jnp.int32, (T, E), 1)
    top1 = jnp.max(logits, axis=1, keepdims=True)
    i1 = jnp.min(jnp.where(logits == top1, iota, E), axis=1, keepdims=True)
    masked = jnp.where(iota == i1, -jnp.inf, logits)
    top2 = jnp.max(masked, axis=1, keepdims=True)
    i2 = jnp.min(jnp.where(masked == top2, iota, E), axis=1, keepdims=True)
    t = jnp.exp(top2 - top1)
    p1 = 1.0 / (1.0 + t)
    p2 = t / (1.0 + t)
    probs_ref[...] = jnp.where(iota == i1, p1, jnp.where(iota == i2, p2, 0.0))
    iota2 = jax.lax.broadcasted_iota(jnp.int32, idx_ref.shape, 1)
    idx_ref[...] = jnp.where(iota2 == 0, i1, i2)


def kernel(x, W):
    B, S, H = x.shape
    E = W.shape[0]
    K = 2
    N = B * S
    T = 2048
    Hh = H // 2
    xf = x.reshape(N, H)
    wt = W.T

    gate, probs, idx = pl.pallas_call(
        _router_kernel,
        grid=(N // T,),
        in_specs=[
            pl.BlockSpec((T, Hh), lambda i: (i, 0)),
            pl.BlockSpec((T, Hh), lambda i: (i, 1)),
            pl.BlockSpec((Hh, E), lambda i: (0, 0)),
            pl.BlockSpec((Hh, E), lambda i: (1, 0)),
        ],
        out_specs=[
            pl.BlockSpec((T, E), lambda i: (i, 0)),
            pl.BlockSpec((T, E), lambda i: (i, 0)),
            pl.BlockSpec((T, K), lambda i: (i, 0)),
        ],
        out_shape=[
            jax.ShapeDtypeStruct((N, E), jnp.float32),
            jax.ShapeDtypeStruct((N, E), jnp.float32),
            jax.ShapeDtypeStruct((N, K), jnp.int32),
        ],
        compiler_params=pltpu.CompilerParams(
            dimension_semantics=("arbitrary",),
        ),
    )(xf, xf, wt, wt)
    return probs.reshape(B, S, E), idx.reshape(B, S, K), gate


# speedup vs baseline: 1.0147x; 1.0147x over previous
import jax
import jax.numpy as jnp
from jax.experimental import pallas as pl
from jax.experimental.pallas import tpu as pltpu


def _router_kernel(x_ref, w_ref, gate_ref, probs_ref, idx_ref):
    T, E = gate_ref.shape
    logits = jnp.dot(x_ref[...], w_ref[...], preferred_element_type=jnp.float32)
    gate_ref[...] = logits
    lt = logits.T
    iota = jax.lax.broadcasted_iota(jnp.int32, (E, T), 0)
    top1 = jnp.max(lt, axis=0, keepdims=True)
    i1 = jnp.min(jnp.where(lt == top1, iota, E), axis=0, keepdims=True)
    masked = jnp.where(iota == i1, -jnp.inf, lt)
    top2 = jnp.max(masked, axis=0, keepdims=True)
    i2 = jnp.min(jnp.where(masked == top2, iota, E), axis=0, keepdims=True)
    t = jnp.exp(top2 - top1)
    p1 = 1.0 / (1.0 + t)
    p2 = t / (1.0 + t)
    probs_t = jnp.where(iota == i1, p1, jnp.where(iota == i2, p2, 0.0))
    probs_ref[...] = probs_t.T
    idx_t = jnp.where(iota == 0, i1, jnp.where(iota == 1, i2, 0))
    idx_ref[...] = idx_t.T[:, : idx_ref.shape[1]]


def kernel(x, W):
    B, S, H = x.shape
    E = W.shape[0]
    K = 2
    N = B * S
    T = 2048
    xf = x.reshape(N, H)
    wt = W.T

    gate, probs, idx = pl.pallas_call(
        _router_kernel,
        grid=(N // T,),
        in_specs=[
            pl.BlockSpec((T, H), lambda i: (i, 0)),
            pl.BlockSpec((H, E), lambda i: (0, 0)),
        ],
        out_specs=[
            pl.BlockSpec((T, E), lambda i: (i, 0)),
            pl.BlockSpec((T, E), lambda i: (i, 0)),
            pl.BlockSpec((T, K), lambda i: (i, 0)),
        ],
        out_shape=[
            jax.ShapeDtypeStruct((N, E), jnp.float32),
            jax.ShapeDtypeStruct((N, E), jnp.float32),
            jax.ShapeDtypeStruct((N, K), jnp.int32),
        ],
        compiler_params=pltpu.CompilerParams(
            dimension_semantics=("arbitrary",),
        ),
    )(xf, wt)
    return probs.reshape(B, S, E), idx.reshape(B, S, K), gate
